# baseline (speedup 1.0000x reference)
import functools

import jax
import jax.numpy as jnp
from jax import lax
from jax.experimental import pallas as pl
from jax.experimental.pallas import tpu as pltpu
from jax.experimental.pallas import tpu_sc as plsc

N = 10000
NPAD = 10240
F = 128
NC, NS, L = 2, 16, 16
NW = NC * NS
E = 320000
CH = 128
KCH = 80
EPT = KCH * CH
EPAD = NW * EPT

_mesh = plsc.VectorSubcoreMesh(core_axis_name="c", subcore_axis_name="s")


@functools.partial(
    pl.kernel,
    out_type=jax.ShapeDtypeStruct((NW, NPAD), jnp.float32),
    mesh=_mesh,
    compiler_params=pltpu.CompilerParams(needs_layout_passes=False),
    scratch_types=[
        pltpu.VMEM((KCH, CH), jnp.int32),
        pltpu.VMEM((NPAD,), jnp.float32),
    ],
)
def _deg_kernel(dst_hbm, out_hbm, dst_v, deg_v):
    cid = lax.axis_index("c")
    sid = lax.axis_index("s")
    wid = sid * NC + cid

    zeros16 = jnp.zeros((L,), jnp.float32)

    def zbody(i, _):
        deg_v[pl.ds(i * L, L)] = zeros16
        return 0

    lax.fori_loop(0, NPAD // L, zbody, 0)

    pltpu.sync_copy(dst_hbm.at[wid], dst_v)

    ones16 = jnp.ones((L,), jnp.float32)

    def ebody(j, _):
        for k in range(CH // L):
            idx = dst_v[j, pl.ds(k * L, L)]
            plsc.addupdate_scatter(deg_v, [idx], ones16)
        return 0

    lax.fori_loop(0, KCH, ebody, 0)

    pltpu.sync_copy(deg_v, out_hbm.at[wid])


WROWS = NPAD // NS


@functools.partial(
    pl.kernel,
    out_type=jax.ShapeDtypeStruct((NC, NPAD, F), jnp.float32),
    mesh=_mesh,
    compiler_params=pltpu.CompilerParams(needs_layout_passes=False),
    scratch_types=[
        pltpu.VMEM((KCH, CH), jnp.int32),
        pltpu.VMEM((KCH, CH), jnp.int32),
        pltpu.VMEM((CH, F), jnp.float32),
        pltpu.VMEM_SHARED((NPAD, F), jnp.float32),
        pltpu.SemaphoreType.DMA,
    ],
)
def _msg_kernel(g_hbm, src_hbm, dst_hbm, out_hbm,
                src_v, dst_v, rows_v, acc_sh, gsem):
    cid = lax.axis_index("c")
    sid = lax.axis_index("s")
    wid = sid * NC + cid

    zeros16 = jnp.zeros((L,), jnp.float32)

    def zbody(i, _):
        rows_v[i // (F // L), pl.ds((i % (F // L)) * L, L)] = zeros16
        return 0

    lax.fori_loop(0, CH * F // L, zbody, 0)
    for q in range(WROWS // CH):
        pltpu.sync_copy(rows_v, acc_sh.at[pl.ds(sid * WROWS + q * CH, CH)])
    plsc.subcore_barrier()

    pltpu.sync_copy(src_hbm.at[wid], src_v)
    pltpu.sync_copy(dst_hbm.at[wid], dst_v)

    def body(j, _):
        pltpu.async_copy(g_hbm.at[src_v.at[j]], rows_v, gsem).wait()
        pltpu.sync_copy(rows_v, acc_sh.at[dst_v.at[j]], add=True)
        return 0

    lax.fori_loop(0, KCH, body, 0)
    plsc.subcore_barrier()

    for q in range(WROWS // CH):
        base = sid * WROWS + q * CH
        pltpu.sync_copy(acc_sh.at[pl.ds(base, CH)], rows_v)
        pltpu.sync_copy(rows_v, out_hbm.at[cid, pl.ds(base, CH)])


def _dinv_col(degp):
    ones = jnp.ones((NW, 1), jnp.float32)
    deg = lax.dot_general(degp, ones, (((0,), (0,)), ((), ())),
                          preferred_element_type=jnp.float32)
    return lax.rsqrt(deg + 1.0)


def _tc_first(x_ref, w_ref, degp_ref, g_ref):
    dinv = _dinv_col(degp_ref[...])
    h = jnp.dot(x_ref[...], w_ref[...], preferred_element_type=jnp.float32)
    g_ref[...] = dinv * h


def _tc_mid(acc_ref, g1_ref, degp_ref, w2_ref, b1_ref, g2_ref):
    dinv = _dinv_col(degp_ref[...])
    out1 = dinv * (acc_ref[0] + acc_ref[1] + g1_ref[...]) + b1_ref[...]
    h2 = jnp.maximum(out1, 0.0)
    g2_ref[...] = dinv * jnp.dot(h2, w2_ref[...],
                                 preferred_element_type=jnp.float32)


def _tc_last(acc_ref, g2_ref, degp_ref, b2_ref, out_ref):
    dinv = _dinv_col(degp_ref[...])
    out_ref[...] = dinv * (acc_ref[0] + acc_ref[1] + g2_ref[...]) + b2_ref[...]


def kernel(x, edge_index, W1, b1, W2, b2):
    src = edge_index[0].astype(jnp.int32)
    dst = edge_index[1].astype(jnp.int32)
    pad = EPAD - E
    src_p = jnp.concatenate([src, jnp.zeros((pad,), jnp.int32)])
    dst_p = jnp.concatenate([dst, jnp.full((pad,), N, jnp.int32)])
    src_r = src_p.reshape(NW, KCH, CH)
    dst_r = dst_p.reshape(NW, KCH, CH)

    x_p = jnp.zeros((NPAD, F), jnp.float32).at[:N].set(x)
    b1r = b1.reshape(1, F)
    b2r = b2.reshape(1, F)

    degp = _deg_kernel(dst_r)

    g1 = pl.pallas_call(
        _tc_first,
        out_shape=jax.ShapeDtypeStruct((NPAD, F), jnp.float32),
    )(x_p, W1, degp)

    acc1 = _msg_kernel(g1, src_r, dst_r)

    g2 = pl.pallas_call(
        _tc_mid,
        out_shape=jax.ShapeDtypeStruct((NPAD, F), jnp.float32),
    )(acc1, g1, degp, W2, b1r)

    acc2 = _msg_kernel(g2, src_r, dst_r)

    out = pl.pallas_call(
        _tc_last,
        out_shape=jax.ShapeDtypeStruct((NPAD, F), jnp.float32),
    )(acc2, g2, degp, b2r)

    return out[:N]


# speedup vs baseline: 2.7636x; 2.7636x over previous
import functools

import jax
import jax.numpy as jnp
from jax import lax
from jax.experimental import pallas as pl
from jax.experimental.pallas import tpu as pltpu
from jax.experimental.pallas import tpu_sc as plsc

N = 10000
NPAD = 10240
F = 128
NC, NS, L = 2, 16, 16
NW = NC * NS
E = 320000
CH = 128
KCH = 80
EPT = KCH * CH
EPAD = NW * EPT

_mesh = plsc.VectorSubcoreMesh(core_axis_name="c", subcore_axis_name="s")


@functools.partial(
    pl.kernel,
    out_type=jax.ShapeDtypeStruct((NW, NPAD), jnp.float32),
    mesh=_mesh,
    compiler_params=pltpu.CompilerParams(needs_layout_passes=False),
    scratch_types=[
        pltpu.VMEM((KCH, CH), jnp.int32),
        pltpu.VMEM((NPAD,), jnp.float32),
    ],
)
def _deg_kernel(dst_hbm, out_hbm, dst_v, deg_v):
    cid = lax.axis_index("c")
    sid = lax.axis_index("s")
    wid = sid * NC + cid

    zeros16 = jnp.zeros((L,), jnp.float32)

    def zbody(i, _):
        deg_v[pl.ds(i * L, L)] = zeros16
        return 0

    lax.fori_loop(0, NPAD // L, zbody, 0)

    pltpu.sync_copy(dst_hbm.at[wid], dst_v)

    ones16 = jnp.ones((L,), jnp.float32)

    def ebody(j, _):
        for k in range(CH // L):
            idx = dst_v[j, pl.ds(k * L, L)]
            plsc.addupdate_scatter(deg_v, [idx], ones16)
        return 0

    lax.fori_loop(0, KCH, ebody, 0)

    pltpu.sync_copy(deg_v, out_hbm.at[wid])


WROWS = NPAD // NS


@functools.partial(
    pl.kernel,
    out_type=jax.ShapeDtypeStruct((NC, NPAD, F), jnp.float32),
    mesh=_mesh,
    compiler_params=pltpu.CompilerParams(needs_layout_passes=False),
    scratch_types=[
        pltpu.VMEM((KCH, CH), jnp.int32),
        pltpu.VMEM((KCH, CH), jnp.int32),
        pltpu.VMEM((CH, F), jnp.float32),
        pltpu.VMEM_SHARED((NPAD, F), jnp.float32),
        pltpu.SemaphoreType.DMA,
    ],
)
def _msg_kernel(g_hbm, src_hbm, dst_hbm, out_hbm,
                src_v, dst_v, rows_v, acc_sh, gsem):
    cid = lax.axis_index("c")
    sid = lax.axis_index("s")
    wid = sid * NC + cid

    zeros16 = jnp.zeros((L,), jnp.float32)

    def zbody(i, _):
        rows_v[i // (F // L), pl.ds((i % (F // L)) * L, L)] = zeros16
        return 0

    lax.fori_loop(0, CH * F // L, zbody, 0)
    for q in range(WROWS // CH):
        pltpu.sync_copy(rows_v, acc_sh.at[pl.ds(sid * WROWS + q * CH, CH)])
    plsc.subcore_barrier()

    pltpu.sync_copy(src_hbm.at[wid], src_v)
    pltpu.sync_copy(dst_hbm.at[wid], dst_v)

    def body(j, _):
        pltpu.async_copy(g_hbm.at[src_v.at[j]], rows_v, gsem).wait()
        pltpu.sync_copy(rows_v, acc_sh.at[dst_v.at[j]], add=True)
        return 0

    lax.fori_loop(0, KCH, body, 0)
    plsc.subcore_barrier()

    for q in range(WROWS // CH):
        base = sid * WROWS + q * CH
        pltpu.sync_copy(acc_sh.at[pl.ds(base, CH)], rows_v)
        pltpu.sync_copy(rows_v, out_hbm.at[cid, pl.ds(base, CH)])


def _dinv_col(degp):
    ones = jnp.ones((NW, 1), jnp.float32)
    deg = lax.dot_general(degp, ones, (((0,), (0,)), ((), ())),
                          preferred_element_type=jnp.float32)
    return lax.rsqrt(deg + 1.0)


def _tc_first(x_ref, w_ref, degp_ref, g_ref):
    dinv = _dinv_col(degp_ref[...])
    h = jnp.dot(x_ref[...], w_ref[...], preferred_element_type=jnp.float32)
    g_ref[...] = dinv * h


def _tc_mid(acc_ref, g1_ref, degp_ref, w2_ref, b1_ref, g2_ref):
    dinv = _dinv_col(degp_ref[...])
    out1 = dinv * (acc_ref[0] + acc_ref[1] + g1_ref[...]) + b1_ref[...]
    h2 = jnp.maximum(out1, 0.0)
    g2_ref[...] = dinv * jnp.dot(h2, w2_ref[...],
                                 preferred_element_type=jnp.float32)


def _tc_last(acc_ref, g2_ref, degp_ref, b2_ref, out_ref):
    dinv = _dinv_col(degp_ref[...])
    out_ref[...] = dinv * (acc_ref[0] + acc_ref[1] + g2_ref[...]) + b2_ref[...]


def kernel(x, edge_index, W1, b1, W2, b2):
    src = edge_index[0].astype(jnp.int32)
    dst = edge_index[1].astype(jnp.int32)
    pad = EPAD - E
    padidx = N + (jnp.arange(pad, dtype=jnp.int32) % (NPAD - N))
    src_p = jnp.concatenate([src, padidx])
    dst_p = jnp.concatenate([dst, padidx])
    src_r = src_p.reshape(NW, KCH, CH)
    dst_r = dst_p.reshape(NW, KCH, CH)

    x_p = jnp.zeros((NPAD, F), jnp.float32).at[:N].set(x)
    b1r = b1.reshape(1, F)
    b2r = b2.reshape(1, F)

    degp = _deg_kernel(dst_r)

    g1 = pl.pallas_call(
        _tc_first,
        out_shape=jax.ShapeDtypeStruct((NPAD, F), jnp.float32),
    )(x_p, W1, degp)

    acc1 = _msg_kernel(g1, src_r, dst_r)

    g2 = pl.pallas_call(
        _tc_mid,
        out_shape=jax.ShapeDtypeStruct((NPAD, F), jnp.float32),
    )(acc1, g1, degp, W2, b1r)

    acc2 = _msg_kernel(g2, src_r, dst_r)

    out = pl.pallas_call(
        _tc_last,
        out_shape=jax.ShapeDtypeStruct((NPAD, F), jnp.float32),
    )(acc2, g2, degp, b2r)

    return out[:N]


# speedup vs baseline: 3.5394x; 1.2807x over previous
import functools

import jax
import jax.numpy as jnp
from jax import lax
from jax.experimental import pallas as pl
from jax.experimental.pallas import tpu as pltpu
from jax.experimental.pallas import tpu_sc as plsc

N = 10000
NPAD = 10240
F = 128
NC, NS, L = 2, 16, 16
NW = NC * NS
E = 320000
CH = 128
KCH = 80
EPT = KCH * CH
EPAD = NW * EPT

_mesh = plsc.VectorSubcoreMesh(core_axis_name="c", subcore_axis_name="s")


@functools.partial(
    pl.kernel,
    out_type=jax.ShapeDtypeStruct((NW, NPAD), jnp.float32),
    mesh=_mesh,
    compiler_params=pltpu.CompilerParams(needs_layout_passes=False),
    scratch_types=[
        pltpu.VMEM((KCH, CH), jnp.int32),
        pltpu.VMEM((NPAD,), jnp.float32),
    ],
)
def _deg_kernel(dst_hbm, out_hbm, dst_v, deg_v):
    cid = lax.axis_index("c")
    sid = lax.axis_index("s")
    wid = sid * NC + cid

    zeros16 = jnp.zeros((L,), jnp.float32)

    def zbody(i, _):
        deg_v[pl.ds(i * L, L)] = zeros16
        return 0

    lax.fori_loop(0, NPAD // L, zbody, 0)

    pltpu.sync_copy(dst_hbm.at[wid], dst_v)

    ones16 = jnp.ones((L,), jnp.float32)

    def ebody(j, _):
        for k in range(CH // L):
            idx = dst_v[j, pl.ds(k * L, L)]
            plsc.addupdate_scatter(deg_v, [idx], ones16)
        return 0

    lax.fori_loop(0, KCH, ebody, 0)

    pltpu.sync_copy(deg_v, out_hbm.at[wid])


WROWS = NPAD // NS


@functools.partial(
    pl.kernel,
    out_type=jax.ShapeDtypeStruct((NC, NPAD, F), jnp.float32),
    mesh=_mesh,
    compiler_params=pltpu.CompilerParams(needs_layout_passes=False),
    scratch_types=[
        pltpu.VMEM((2, CH), jnp.int32),
        pltpu.VMEM((KCH, CH), jnp.int32),
        pltpu.VMEM((2, CH, F), jnp.float32),
        pltpu.VMEM_SHARED((NPAD, F), jnp.float32),
        pltpu.SemaphoreType.DMA,
        pltpu.SemaphoreType.DMA,
    ],
)
def _msg_kernel(g_hbm, src_hbm, dst_hbm, out_hbm,
                srcc_v, dst_v, rows_v, acc_sh, gsem, isem):
    cid = lax.axis_index("c")
    sid = lax.axis_index("s")
    wid = sid * NC + cid

    zeros16 = jnp.zeros((L,), jnp.float32)

    def zbody(i, _):
        rows_v[0, i // (F // L), pl.ds((i % (F // L)) * L, L)] = zeros16
        return 0

    lax.fori_loop(0, CH * F // L, zbody, 0)
    for q in range(WROWS // CH):
        pltpu.sync_copy(rows_v.at[0],
                        acc_sh.at[pl.ds(sid * WROWS + q * CH, CH)])
    plsc.subcore_barrier()

    pltpu.sync_copy(dst_hbm.at[wid], dst_v)

    pltpu.sync_copy(src_hbm.at[wid, 0], srcc_v.at[0])
    pltpu.async_copy(g_hbm.at[srcc_v.at[0]], rows_v.at[0], gsem)
    pltpu.async_copy(src_hbm.at[wid, 1], srcc_v.at[1], isem)

    def body(jj, _):
        j0 = jj * 2
        for b in range(2):
            j = j0 + b
            nb = 1 - b
            pltpu.make_async_copy(g_hbm.at[srcc_v.at[b]], rows_v.at[b],
                                  gsem).wait()
            pltpu.make_async_copy(src_hbm.at[wid, 0], srcc_v.at[nb],
                                  isem).wait()
            pltpu.async_copy(g_hbm.at[srcc_v.at[nb]], rows_v.at[nb], gsem)

            @pl.when(j + 2 < KCH)
            def _():
                pltpu.async_copy(src_hbm.at[wid, j + 2], srcc_v.at[b], isem)

            pltpu.sync_copy(rows_v.at[b], acc_sh.at[dst_v.at[j]], add=True)
        return 0

    lax.fori_loop(0, KCH // 2 - 1, body, 0)
    j0 = KCH - 2
    pltpu.make_async_copy(g_hbm.at[srcc_v.at[0]], rows_v.at[0], gsem).wait()
    pltpu.make_async_copy(src_hbm.at[wid, 0], srcc_v.at[1], isem).wait()
    pltpu.async_copy(g_hbm.at[srcc_v.at[1]], rows_v.at[1], gsem)
    pltpu.sync_copy(rows_v.at[0], acc_sh.at[dst_v.at[j0]], add=True)
    pltpu.make_async_copy(g_hbm.at[srcc_v.at[1]], rows_v.at[1], gsem).wait()
    pltpu.sync_copy(rows_v.at[1], acc_sh.at[dst_v.at[j0 + 1]], add=True)
    plsc.subcore_barrier()

    for q in range(WROWS // CH):
        base = sid * WROWS + q * CH
        pltpu.sync_copy(acc_sh.at[pl.ds(base, CH)], rows_v.at[0])
        pltpu.sync_copy(rows_v.at[0], out_hbm.at[cid, pl.ds(base, CH)])


def _dinv_col(degp):
    ones = jnp.ones((NW, 1), jnp.float32)
    deg = lax.dot_general(degp, ones, (((0,), (0,)), ((), ())),
                          preferred_element_type=jnp.float32)
    return lax.rsqrt(deg + 1.0)


def _tc_first(x_ref, w_ref, degp_ref, g_ref):
    dinv = _dinv_col(degp_ref[...])
    h = jnp.dot(x_ref[...], w_ref[...], preferred_element_type=jnp.float32)
    g_ref[...] = dinv * h


def _tc_mid(acc_ref, g1_ref, degp_ref, w2_ref, b1_ref, g2_ref):
    dinv = _dinv_col(degp_ref[...])
    out1 = dinv * (acc_ref[0] + acc_ref[1] + g1_ref[...]) + b1_ref[...]
    h2 = jnp.maximum(out1, 0.0)
    g2_ref[...] = dinv * jnp.dot(h2, w2_ref[...],
                                 preferred_element_type=jnp.float32)


def _tc_last(acc_ref, g2_ref, degp_ref, b2_ref, out_ref):
    dinv = _dinv_col(degp_ref[...])
    out_ref[...] = dinv * (acc_ref[0] + acc_ref[1] + g2_ref[...]) + b2_ref[...]


def kernel(x, edge_index, W1, b1, W2, b2):
    src = edge_index[0].astype(jnp.int32)
    dst = edge_index[1].astype(jnp.int32)
    pad = EPAD - E
    padidx = N + (jnp.arange(pad, dtype=jnp.int32) % (NPAD - N))
    src_p = jnp.concatenate([src, padidx])
    dst_p = jnp.concatenate([dst, padidx])
    src_r = src_p.reshape(NW, KCH, CH)
    dst_r = dst_p.reshape(NW, KCH, CH)

    x_p = jnp.zeros((NPAD, F), jnp.float32).at[:N].set(x)
    b1r = b1.reshape(1, F)
    b2r = b2.reshape(1, F)

    degp = _deg_kernel(dst_r)

    g1 = pl.pallas_call(
        _tc_first,
        out_shape=jax.ShapeDtypeStruct((NPAD, F), jnp.float32),
    )(x_p, W1, degp)

    acc1 = _msg_kernel(g1, src_r, dst_r)

    g2 = pl.pallas_call(
        _tc_mid,
        out_shape=jax.ShapeDtypeStruct((NPAD, F), jnp.float32),
    )(acc1, g1, degp, W2, b1r)

    acc2 = _msg_kernel(g2, src_r, dst_r)

    out = pl.pallas_call(
        _tc_last,
        out_shape=jax.ShapeDtypeStruct((NPAD, F), jnp.float32),
    )(acc2, g2, degp, b2r)

    return out[:N]
